# Initial kernel scaffold; baseline (speedup 1.0000x reference)
import jax
import jax.numpy as jnp
from jax.experimental import pallas as pl


def kernel(inputs, E0, E1, W2, b2, W3, b3, W4, b4, W5, b5, W6, b6, W7, b7):
    raise NotImplementedError("write your pallas kernel here")


# speedup vs baseline: 2.3280x; 2.3280x over previous
import functools

import jax
import jax.numpy as jnp
from jax import lax
from jax.experimental import pallas as pl
from jax.experimental.pallas import tpu as pltpu
from jax.experimental.pallas import tpu_sc as plsc

B = 1024
W = 200
NUM_INPUTS = 8
D = 64
HIST = 150
FUT = W - HIST

NC = 2
NS = 16
NW = NC * NS

CHUNK = 128
ROWS_PER_W = (B * W) // NW
CHUNKS_PER_W = ROWS_PER_W // CHUNK
S_PER_W = B // NW


def _sc_gather(e0_hbm, e1_hbm, idx0_hbm, idx1_hbm, g_hbm, s_hbm,
               idx_v, rows_v, idx0_v, rows0_v, sem):
    wid = lax.axis_index("s") * NC + lax.axis_index("c")

    pltpu.sync_copy(idx0_hbm.at[wid], idx0_v)
    pltpu.async_copy(e0_hbm.at[idx0_v.at[0]], rows0_v, sem).wait()
    pltpu.sync_copy(rows0_v, s_hbm.at[pl.ds(wid * S_PER_W, S_PER_W)])

    pltpu.sync_copy(idx1_hbm.at[wid], idx_v)
    base = wid * ROWS_PER_W

    def body(j, carry):
        pltpu.async_copy(e1_hbm.at[idx_v.at[j]], rows_v, sem).wait()
        pltpu.sync_copy(rows_v, g_hbm.at[pl.ds(base + j * CHUNK, CHUNK)])
        return carry

    lax.fori_loop(0, CHUNKS_PER_W, body, 0)


def _tc_body(xt_ref, g_ref, wm_ref, bm_ref, hist_ref, fut_ref):
    g = g_ref[...]

    def dense(i, tlo, thi):
        xi = xt_ref[i, :, tlo:thi]
        return xi[:, :, None] * wm_ref[i][None, None, :] + bm_ref[i][None, None, :]

    hist_ref[:, :, 0 * D:1 * D] = dense(7, 0, HIST)
    hist_ref[:, :, 1 * D:2 * D] = g[:, :HIST, :]
    hist_ref[:, :, 2 * D:3 * D] = dense(5, 0, HIST)
    hist_ref[:, :, 3 * D:4 * D] = dense(6, 0, HIST)
    hist_ref[:, :, 4 * D:5 * D] = dense(2, 0, HIST)
    hist_ref[:, :, 5 * D:6 * D] = dense(3, 0, HIST)
    hist_ref[:, :, 6 * D:7 * D] = dense(4, 0, HIST)
    fut_ref[:, :, 0 * D:1 * D] = g[:, HIST:, :]
    fut_ref[:, :, 1 * D:2 * D] = dense(5, HIST, W)
    fut_ref[:, :, 2 * D:3 * D] = dense(6, HIST, W)


def kernel(inputs, E0, E1, W2, b2, W3, b3, W4, b4, W5, b5, W6, b6, W7, b7):
    f32 = jnp.float32
    idx0 = inputs[:, 0, 0].astype(jnp.int32).reshape(NW, 1, S_PER_W)
    idx1 = inputs[:, :, 1].astype(jnp.int32).reshape(NW, CHUNKS_PER_W, CHUNK)

    mesh = plsc.VectorSubcoreMesh(core_axis_name="c", subcore_axis_name="s")
    sc = pl.kernel(
        _sc_gather,
        mesh=mesh,
        out_type=[
            jax.ShapeDtypeStruct((B * W, D), f32),
            jax.ShapeDtypeStruct((B, D), f32),
        ],
        scratch_types=[
            pltpu.VMEM((CHUNKS_PER_W, CHUNK), jnp.int32),
            pltpu.VMEM((CHUNK, D), f32),
            pltpu.VMEM((1, S_PER_W), jnp.int32),
            pltpu.VMEM((S_PER_W, D), f32),
            pltpu.SemaphoreType.DMA,
        ],
        compiler_params=pltpu.CompilerParams(use_tc_tiling_on_sc=False),
    )
    g_rows, s_rows = sc(E0, E1, idx0, idx1)

    x_t = jnp.moveaxis(inputs, 2, 0)
    zero = jnp.zeros((1, D), f32)
    wm = jnp.concatenate([zero, zero, W2, W3, W4, W5, W6, W7], axis=0)
    bm = jnp.stack([jnp.zeros((D,), f32)] * 2 + [b2, b3, b4, b5, b6, b7], axis=0)

    bb = 8
    hist_flat, fut_flat = pl.pallas_call(
        _tc_body,
        grid=(B // bb,),
        in_specs=[
            pl.BlockSpec((NUM_INPUTS, bb, W), lambda b: (0, b, 0)),
            pl.BlockSpec((bb, W, D), lambda b: (b, 0, 0)),
            pl.BlockSpec((NUM_INPUTS, D), lambda b: (0, 0)),
            pl.BlockSpec((NUM_INPUTS, D), lambda b: (0, 0)),
        ],
        out_specs=[
            pl.BlockSpec((bb, HIST, 7 * D), lambda b: (b, 0, 0)),
            pl.BlockSpec((bb, FUT, 3 * D), lambda b: (b, 0, 0)),
        ],
        out_shape=[
            jax.ShapeDtypeStruct((B, HIST, 7 * D), f32),
            jax.ShapeDtypeStruct((B, FUT, 3 * D), f32),
        ],
    )(x_t, g_rows.reshape(B, W, D), wm, bm)

    static = s_rows.reshape(B, 1, D)
    historical = hist_flat.reshape(B, HIST, 7, D)
    future = fut_flat.reshape(B, FUT, 3, D)
    return (static, historical, future)
